# baseline (speedup 1.0000x reference)
import functools

import jax
import jax.numpy as jnp
from jax import lax
from jax.experimental import pallas as pl
from jax.experimental.pallas import tpu as pltpu
from jax.experimental.pallas import tpu_sc as plsc

N = 50000
NP = 50048
RPT = NP // 16
E = 800000
EPAD = 819200
NCH = EPAD // 256
DUMMY = 50040


def _mesh():
    return plsc.VectorSubcoreMesh(core_axis_name="c", subcore_axis_name="s",
                                  num_cores=2, num_subcores=16)


_SC_PARAMS = pltpu.CompilerParams(use_tc_tiling_on_sc=False)


def _agg_body(*refs, phases, ntab):
    tables = refs[:ntab]
    src_r, dst_r, zeros, out = refs[ntab:ntab + 4]
    (ibs0, ibd0, ibs1, ibd1, r0, r1, r2, r3, r4, acc,
     i0s, i1s, g0s, g1s, g2s, g3s, g4s, s0s, s1s, s2s, s3s, s4s) = refs[ntab + 4:]
    c = lax.axis_index("c")
    t = lax.axis_index("s")
    ibufs = ((ibs0, ibd0), (ibs1, ibd1))
    isems = (i0s, i1s)
    rbufs = (r0, r1, r2, r3, r4)
    gsems = (g0s, g1s, g2s, g3s, g4s)
    ssems = (s0s, s1s, s2s, s3s, s4s)

    for pi, (ti, stacked, spt) in enumerate(phases):
        tbl = tables[ti].at[c] if stacked else tables[ti]
        base = t * spt if stacked else c * 320 + t * spt

        pltpu.sync_copy(zeros, acc.at[pl.ds(t * RPT, RPT)])
        plsc.subcore_barrier()

        def ifetch(k, half, sync=False):
            ibs, ibd = ibufs[half]
            if sync:
                pltpu.sync_copy(src_r.at[pl.ds((base + k) * 10, 10)], ibs)
                pltpu.sync_copy(dst_r.at[pl.ds((base + k) * 10, 10)], ibd)
            else:
                pltpu.async_copy(src_r.at[pl.ds((base + k) * 10, 10)], ibs,
                                 isems[half])
                pltpu.async_copy(dst_r.at[pl.ds((base + k) * 10, 10)], ibd,
                                 isems[half])

        def iwait(k, half):
            ibs, ibd = ibufs[half]
            pltpu.make_async_copy(src_r.at[pl.ds((base + k) * 10, 10)], ibs,
                                  isems[half]).wait()
            pltpu.make_async_copy(dst_r.at[pl.ds((base + k) * 10, 10)], ibd,
                                  isems[half]).wait()

        def gather(half, v, b):
            pltpu.async_copy(tbl.at[ibufs[half][0].at[v]], rbufs[b], gsems[b])

        def gwait(half, v, b):
            pltpu.make_async_copy(tbl.at[ibufs[half][0].at[v]], rbufs[b],
                                  gsems[b]).wait()

        def scat(half, v, b):
            pltpu.async_copy(rbufs[b], acc.at[ibufs[half][1].at[v]], ssems[b],
                             add=True)

        def swait(half, v, b):
            pltpu.make_async_copy(rbufs[b], acc.at[ibufs[half][1].at[v]],
                                  ssems[b]).wait()

        ifetch(0, 0, sync=True)
        for v in range(2):
            gather(0, v, v)

        def outer(k2, carry):
            for half in range(2):
                k = 2 * k2 + half
                for uu in range(10):
                    b = uu % 5
                    gwait(half, uu, b)
                    if uu <= 2:
                        @pl.when(k > 0)
                        def _():
                            swait(half, uu, (b + 2) % 5)
                    else:
                        swait(half, uu, (b + 2) % 5)
                    if uu == 3:
                        @pl.when(k < spt - 1)
                        def _():
                            ifetch(k + 1, 1 - half)
                    if uu <= 7:
                        gather(half, uu + 2, (b + 2) % 5)
                    else:
                        @pl.when(k < spt - 1)
                        def _():
                            if uu == 8:
                                iwait(k + 1, 1 - half)
                            gather(1 - half, uu - 8, (b + 2) % 5)
                    scat(half, uu, b)
            return carry

        lax.fori_loop(0, spt // 2, outer, 0)
        for uu in (7, 8, 9):
            swait(1, uu, uu % 5)
        plsc.subcore_barrier()
        pltpu.sync_copy(acc.at[pl.ds(t * RPT, RPT)],
                        out.at[2 * pi + c].at[pl.ds(t * RPT, RPT)])
        plsc.subcore_barrier()


def _agg_layer(tables, src_r, dst_r, zeros32, *, phases):
    ntab = len(tables)
    body = functools.partial(_agg_body, phases=phases, ntab=ntab)
    sem = pltpu.SemaphoreType.DMA
    return pl.kernel(
        body,
        out_type=jax.ShapeDtypeStruct((2 * len(phases), NP, 32), jnp.float32),
        mesh=_mesh(),
        compiler_params=_SC_PARAMS,
        scratch_types=[
            pltpu.VMEM((10, 128), jnp.int32),
            pltpu.VMEM((10, 128), jnp.int32),
            pltpu.VMEM((10, 128), jnp.int32),
            pltpu.VMEM((10, 128), jnp.int32),
            pltpu.VMEM((128, 32), jnp.float32),
            pltpu.VMEM((128, 32), jnp.float32),
            pltpu.VMEM((128, 32), jnp.float32),
            pltpu.VMEM((128, 32), jnp.float32),
            pltpu.VMEM((128, 32), jnp.float32),
            pltpu.VMEM_SHARED((NP, 32), jnp.float32),
            sem, sem, sem, sem, sem, sem, sem, sem, sem, sem, sem, sem,
        ],
    )(*tables, src_r, dst_r, zeros32)


def _deg_body(dst_r, ones, zeros, out, ib0, ib1, ones_v, acc, s0s, s1s):
    c = lax.axis_index("c")
    t = lax.axis_index("s")
    spt = 20
    pltpu.sync_copy(zeros, acc.at[pl.ds(t * RPT, RPT)])
    pltpu.sync_copy(ones, ones_v)
    plsc.subcore_barrier()
    base = c * 320 + t * spt

    def scat(ib, sem):
        for j in range(10):
            pltpu.async_copy(ones_v, acc.at[ib.at[j]], sem, add=True)

    def swait(ib, sem):
        for j in range(10):
            pltpu.make_async_copy(ones_v, acc.at[ib.at[j]], sem).wait()

    def pair(p, carry):
        @pl.when(p > 0)
        def _():
            swait(ib0, s0s)

        pltpu.sync_copy(dst_r.at[pl.ds((base + 2 * p) * 10, 10)], ib0)
        scat(ib0, s0s)

        @pl.when(p > 0)
        def _():
            swait(ib1, s1s)

        pltpu.sync_copy(dst_r.at[pl.ds((base + 2 * p + 1) * 10, 10)], ib1)
        scat(ib1, s1s)
        return carry

    lax.fori_loop(0, spt // 2, pair, 0)
    swait(ib0, s0s)
    swait(ib1, s1s)
    plsc.subcore_barrier()
    pltpu.sync_copy(acc.at[pl.ds(t * RPT, RPT)], out.at[c].at[pl.ds(t * RPT, RPT)])


def _deg_pass(dst_r, ones8, zeros8):
    return pl.kernel(
        _deg_body,
        out_type=jax.ShapeDtypeStruct((2, NP, 8), jnp.float32),
        mesh=_mesh(),
        compiler_params=_SC_PARAMS,
        scratch_types=[
            pltpu.VMEM((10, 128), jnp.int32),
            pltpu.VMEM((10, 128), jnp.int32),
            pltpu.VMEM((128, 8), jnp.float32),
            pltpu.VMEM_SHARED((NP, 8), jnp.float32),
            pltpu.SemaphoreType.DMA,
            pltpu.SemaphoreType.DMA,
        ],
    )(dst_r, ones8, zeros8)


def _l0_body(x_ref, deg_ref, xsa_ref, xsb_ref, s_ref):
    d = deg_ref[0, :, 0:1] + deg_ref[1, :, 0:1]
    s = lax.rsqrt(1.0 + d)
    xs = x_ref[...] * s
    xsa_ref[0] = xs[:, 0:32]
    xsa_ref[1] = xs[:, 32:64]
    xsb_ref[...] = xs[:, 64:96]
    s_ref[...] = s


def _prep(x_pad, deg):
    return pl.pallas_call(
        _l0_body,
        grid=(16,),
        in_specs=[
            pl.BlockSpec((RPT, 96), lambda i: (i, 0)),
            pl.BlockSpec((2, RPT, 8), lambda i: (0, i, 0)),
        ],
        out_specs=[
            pl.BlockSpec((2, RPT, 32), lambda i: (0, i, 0)),
            pl.BlockSpec((RPT, 32), lambda i: (i, 0)),
            pl.BlockSpec((RPT, 1), lambda i: (i, 0)),
        ],
        out_shape=[
            jax.ShapeDtypeStruct((2, NP, 32), jnp.float32),
            jax.ShapeDtypeStruct((NP, 32), jnp.float32),
            jax.ShapeDtypeStruct((NP, 1), jnp.float32),
        ],
    )(x_pad, deg)


def _u_slices(agg_ref, hsa_ref, hsb_ref, s):
    u0 = (agg_ref[0] + hsa_ref[0]) * s
    u1 = (agg_ref[1] + hsa_ref[1]) * s
    u2 = (agg_ref[2] + agg_ref[3] + hsb_ref[...]) * s
    return (u0, u1, u2)


def _matmul_sliced(us, w_ref, b_ref, width, rows):
    acc = jnp.zeros((rows, width), jnp.float32) + b_ref[...]
    for j, u in enumerate(us):
        acc = acc + jnp.dot(u, w_ref[pl.ds(j * 32, 32)],
                            preferred_element_type=jnp.float32)
    return acc


def _l1_body(agg_ref, hsa_ref, hsb_ref, s_ref, w_ref, b_ref,
             outa_ref, outb_ref):
    i = pl.program_id(0)
    s = s_ref[...]
    us = _u_slices(agg_ref, hsa_ref, hsb_ref, s)
    h = jnp.maximum(_matmul_sliced(us, w_ref, b_ref, 96, RPT), 0.0) * s
    rid = lax.broadcasted_iota(jnp.int32, (RPT, 1), 0) + i * RPT
    h = jnp.where(rid < N, h, 0.0)
    outa_ref[0] = h[:, 0:32]
    outa_ref[1] = h[:, 32:64]
    outb_ref[...] = h[:, 64:96]


def _layer1(agg, hsa, hsb, s, w, b):
    spec4 = pl.BlockSpec((4, RPT, 32), lambda i: (0, i, 0))
    spec2 = pl.BlockSpec((2, RPT, 32), lambda i: (0, i, 0))
    spec1 = pl.BlockSpec((RPT, 32), lambda i: (i, 0))
    return pl.pallas_call(
        _l1_body,
        grid=(16,),
        in_specs=[
            spec4, spec2, spec1,
            pl.BlockSpec((RPT, 1), lambda i: (i, 0)),
            pl.BlockSpec((96, 96), lambda i: (0, 0)),
            pl.BlockSpec((1, 96), lambda i: (0, 0)),
        ],
        out_specs=[spec2, spec1],
        out_shape=[
            jax.ShapeDtypeStruct((2, NP, 32), jnp.float32),
            jax.ShapeDtypeStruct((NP, 32), jnp.float32),
        ],
    )(agg, hsa, hsb, s, w, b)


def _l2_body(agg_ref, hsa_ref, hsb_ref, s_ref, w_ref, b_ref,
             outa1_ref, outa2_ref, outb_ref):
    i = pl.program_id(0)
    s = s_ref[...]
    us = _u_slices(agg_ref, hsa_ref, hsb_ref, s)
    h = jnp.maximum(_matmul_sliced(us, w_ref, b_ref, 160, RPT), 0.0) * s
    rid = lax.broadcasted_iota(jnp.int32, (RPT, 1), 0) + i * RPT
    h = jnp.where(rid < N, h, 0.0)
    outa1_ref[0] = h[:, 0:32]
    outa1_ref[1] = h[:, 32:64]
    outa2_ref[0] = h[:, 64:96]
    outa2_ref[1] = h[:, 96:128]
    outb_ref[...] = h[:, 128:160]


def _layer2(agg, hsa, hsb, s, w, b):
    spec4 = pl.BlockSpec((4, RPT, 32), lambda i: (0, i, 0))
    spec2 = pl.BlockSpec((2, RPT, 32), lambda i: (0, i, 0))
    spec1 = pl.BlockSpec((RPT, 32), lambda i: (i, 0))
    return pl.pallas_call(
        _l2_body,
        grid=(16,),
        in_specs=[
            spec4, spec2, spec1,
            pl.BlockSpec((RPT, 1), lambda i: (i, 0)),
            pl.BlockSpec((96, 160), lambda i: (0, 0)),
            pl.BlockSpec((1, 160), lambda i: (0, 0)),
        ],
        out_specs=[spec2, spec2, spec1],
        out_shape=[
            jax.ShapeDtypeStruct((2, NP, 32), jnp.float32),
            jax.ShapeDtypeStruct((2, NP, 32), jnp.float32),
            jax.ShapeDtypeStruct((NP, 32), jnp.float32),
        ],
    )(agg, hsa, hsb, s, w, b)


PBLK = 400


def _l3_body(batch_sm, agg_ref, h1_ref, h2_ref, hb_ref,
             s_ref, b2d_ref, w_ref, b_ref, out_ref):
    i = pl.program_id(0)

    @pl.when(i == 0)
    def _():
        out_ref[...] = jnp.zeros_like(out_ref)

    s = s_ref[...]
    us = (
        (agg_ref[0] + h1_ref[0]) * s,
        (agg_ref[1] + h1_ref[1]) * s,
        (agg_ref[2] + h2_ref[0]) * s,
        (agg_ref[3] + h2_ref[1]) * s,
        (agg_ref[4] + agg_ref[5] + hb_ref[...]) * s,
    )
    h3 = jnp.maximum(_matmul_sliced(us, w_ref, b_ref, 320, PBLK), 0.0)

    g0 = batch_sm[i * PBLK]
    g1 = batch_sm[i * PBLK + PBLK - 1]
    bids = b2d_ref[...]

    def upd(g, carry):
        gid = g0 + g
        m = (bids == gid)
        v = jnp.max(jnp.where(m, h3, 0.0), axis=0, keepdims=True)
        out_ref[pl.ds(gid, 1), :] = jnp.maximum(out_ref[pl.ds(gid, 1), :], v)
        return carry

    lax.fori_loop(0, g1 - g0 + 1, upd, 0)


def _layer3_pool(batch, agg, h1, h2, hb, s, batch2d, w, b):
    spec6 = pl.BlockSpec((6, PBLK, 32), lambda i, sm: (0, i, 0))
    spec2 = pl.BlockSpec((2, PBLK, 32), lambda i, sm: (0, i, 0))
    spec1 = pl.BlockSpec((PBLK, 32), lambda i, sm: (i, 0))
    grid_spec = pltpu.PrefetchScalarGridSpec(
        num_scalar_prefetch=1,
        grid=(N // PBLK,),
        in_specs=[
            spec6, spec2, spec2, spec1,
            pl.BlockSpec((PBLK, 1), lambda i, sm: (i, 0)),
            pl.BlockSpec((PBLK, 1), lambda i, sm: (i, 0)),
            pl.BlockSpec((160, 320), lambda i, sm: (0, 0)),
            pl.BlockSpec((1, 320), lambda i, sm: (0, 0)),
        ],
        out_specs=pl.BlockSpec((512, 320), lambda i, sm: (0, 0)),
    )
    return pl.pallas_call(
        _l3_body,
        grid_spec=grid_spec,
        out_shape=jax.ShapeDtypeStruct((512, 320), jnp.float32),
    )(batch, agg, h1, h2, hb, s, batch2d, w, b)


def _mlp_body(p_ref, w1_ref, b1_ref, w2_ref, b2_ref, out_ref):
    g = jnp.maximum(jnp.dot(p_ref[...], w1_ref[...],
                            preferred_element_type=jnp.float32) + b1_ref[...], 0.0)
    out_ref[...] = jnp.dot(g, w2_ref[...],
                           preferred_element_type=jnp.float32) + b2_ref[...]


def _mlp(p, w1, b1, w2, b2):
    return pl.pallas_call(
        _mlp_body,
        out_shape=jax.ShapeDtypeStruct((512, 128), jnp.float32),
    )(p, w1, b1, w2, b2)


def kernel(x, edge_index, batch, W1, b1, W2, b2, W3, b3, Wg1, bg1, Wg2, bg2):
    f32 = jnp.float32
    src = edge_index[0].astype(jnp.int32)
    dst = edge_index[1].astype(jnp.int32)
    npad = EPAD - E
    src_r = jnp.concatenate(
        [src, jnp.full((npad,), DUMMY, jnp.int32)]).reshape(EPAD // 128, 128)
    dst_r = jnp.concatenate(
        [dst, jnp.full((npad,), DUMMY, jnp.int32)]).reshape(EPAD // 128, 128)

    zeros32 = jnp.zeros((RPT, 32), f32)
    zeros8 = jnp.zeros((RPT, 8), f32)
    ones8 = jnp.concatenate(
        [jnp.ones((128, 1), f32), jnp.zeros((128, 7), f32)], axis=1)

    x_pad = jnp.zeros((NP, 96), f32).at[:N, :75].set(x)
    W1p = jnp.zeros((96, 96), f32).at[:75, :75].set(W1)
    b1p = jnp.zeros((1, 96), f32).at[0, :75].set(b1)
    W2p = jnp.zeros((96, 160), f32).at[:75, :150].set(W2)
    b2p = jnp.zeros((1, 160), f32).at[0, :150].set(b2)
    W3p = jnp.zeros((160, 320), f32).at[:150, :300].set(W3)
    b3p = jnp.zeros((1, 320), f32).at[0, :300].set(b3)
    Wg1p = jnp.zeros((320, 1024), f32).at[:300].set(Wg1)
    bg1r = bg1.reshape(1, 1024)
    bg2r = bg2.reshape(1, 128)
    batch2d = batch.astype(jnp.int32).reshape(N, 1)

    deg = _deg_pass(dst_r, ones8, zeros8)
    xsa, xsb, s = _prep(x_pad, deg)

    lay12 = [(0, True, 40), (1, False, 20)]
    agg1 = _agg_layer((xsa, xsb), src_r, dst_r, zeros32, phases=lay12)
    h1sa, h1sb = _layer1(agg1, xsa, xsb, s, W1p, b1p)

    agg2 = _agg_layer((h1sa, h1sb), src_r, dst_r, zeros32, phases=lay12)
    h2sa1, h2sa2, h2sb = _layer2(agg2, h1sa, h1sb, s, W2p, b2p)

    agg3 = _agg_layer((h2sa1, h2sa2, h2sb), src_r, dst_r, zeros32,
                      phases=[(0, True, 40), (1, True, 40), (2, False, 20)])

    pooled = _layer3_pool(batch.astype(jnp.int32), agg3,
                          h2sa1, h2sa2, h2sb, s, batch2d, W3p, b3p)
    return _mlp(pooled, Wg1p, bg1r, Wg2, bg2r)


# speedup vs baseline: 1.0495x; 1.0495x over previous
import functools

import jax
import jax.numpy as jnp
from jax import lax
from jax.experimental import pallas as pl
from jax.experimental.pallas import tpu as pltpu
from jax.experimental.pallas import tpu_sc as plsc

N = 50000
NP = 50048
RPT = NP // 16
E = 800000
EPAD = 819200
NCH = EPAD // 256
DUMMY = 50040


def _mesh():
    return plsc.VectorSubcoreMesh(core_axis_name="c", subcore_axis_name="s",
                                  num_cores=2, num_subcores=16)


_SC_PARAMS = pltpu.CompilerParams(use_tc_tiling_on_sc=False)


def _agg_body(*refs, phases, ntab):
    tables = refs[:ntab]
    src_r, dst_r, zeros, out = refs[ntab:ntab + 4]
    (ibs0, ibd0, ibs1, ibd1, r0, r1, r2, r3, r4, acc,
     i0s, i1s, g0s, g1s, g2s, g3s, g4s, s0s, s1s, s2s, s3s, s4s) = refs[ntab + 4:]
    c = lax.axis_index("c")
    t = lax.axis_index("s")
    ibufs = ((ibs0, ibd0), (ibs1, ibd1))
    isems = (i0s, i1s)
    rbufs = (r0, r1, r2, r3, r4)
    gsems = (g0s, g1s, g2s, g3s, g4s)
    ssems = (s0s, s1s, s2s, s3s, s4s)

    for pi, (ti, stacked, spt) in enumerate(phases):
        tbl = tables[ti].at[c] if stacked else tables[ti]
        base = t * spt if stacked else c * 320 + t * spt

        pltpu.sync_copy(zeros, acc.at[pl.ds(t * RPT, RPT)])
        plsc.subcore_barrier()

        def ifetch(k, half, sync=False):
            ibs, ibd = ibufs[half]
            if sync:
                pltpu.sync_copy(src_r.at[pl.ds((base + k) * 10, 10)], ibs)
                pltpu.sync_copy(dst_r.at[pl.ds((base + k) * 10, 10)], ibd)
            else:
                pltpu.async_copy(src_r.at[pl.ds((base + k) * 10, 10)], ibs,
                                 isems[half])
                pltpu.async_copy(dst_r.at[pl.ds((base + k) * 10, 10)], ibd,
                                 isems[half])

        def iwait(k, half):
            ibs, ibd = ibufs[half]
            pltpu.make_async_copy(src_r.at[pl.ds((base + k) * 10, 10)], ibs,
                                  isems[half]).wait()
            pltpu.make_async_copy(dst_r.at[pl.ds((base + k) * 10, 10)], ibd,
                                  isems[half]).wait()

        def gather(half, v, b):
            pltpu.async_copy(tbl.at[ibufs[half][0].at[v]], rbufs[b], gsems[b])

        def gwait(half, v, b):
            pltpu.make_async_copy(tbl.at[ibufs[half][0].at[v]], rbufs[b],
                                  gsems[b]).wait()

        def scat(half, v, b):
            pltpu.async_copy(rbufs[b], acc.at[ibufs[half][1].at[v]], ssems[b],
                             add=True)

        def swait(half, v, b):
            pltpu.make_async_copy(rbufs[b], acc.at[ibufs[half][1].at[v]],
                                  ssems[b]).wait()

        ifetch(0, 0, sync=True)
        for v in range(4):
            gather(0, v, v)

        def outer(k2, carry):
            for half in range(2):
                k = 2 * k2 + half
                for uu in range(10):
                    b = uu % 5
                    gwait(half, uu, b)
                    if uu == 0:
                        @pl.when(k > 0)
                        def _():
                            swait(half, uu, (b + 4) % 5)
                    else:
                        swait(half, uu, (b + 4) % 5)
                    if uu == 1:
                        @pl.when(k < spt - 1)
                        def _():
                            ifetch(k + 1, 1 - half)
                    if uu <= 5:
                        gather(half, uu + 4, (b + 4) % 5)
                    else:
                        @pl.when(k < spt - 1)
                        def _():
                            if uu == 6:
                                iwait(k + 1, 1 - half)
                            gather(1 - half, uu - 6, (b + 4) % 5)
                    scat(half, uu, b)
            return carry

        lax.fori_loop(0, spt // 2, outer, 0)
        swait(1, 9, 4)
        plsc.subcore_barrier()
        pltpu.sync_copy(acc.at[pl.ds(t * RPT, RPT)],
                        out.at[2 * pi + c].at[pl.ds(t * RPT, RPT)])
        plsc.subcore_barrier()


def _agg_layer(tables, src_r, dst_r, zeros32, *, phases):
    ntab = len(tables)
    body = functools.partial(_agg_body, phases=phases, ntab=ntab)
    sem = pltpu.SemaphoreType.DMA
    return pl.kernel(
        body,
        out_type=jax.ShapeDtypeStruct((2 * len(phases), NP, 32), jnp.float32),
        mesh=_mesh(),
        compiler_params=_SC_PARAMS,
        scratch_types=[
            pltpu.VMEM((10, 128), jnp.int32),
            pltpu.VMEM((10, 128), jnp.int32),
            pltpu.VMEM((10, 128), jnp.int32),
            pltpu.VMEM((10, 128), jnp.int32),
            pltpu.VMEM((128, 32), jnp.float32),
            pltpu.VMEM((128, 32), jnp.float32),
            pltpu.VMEM((128, 32), jnp.float32),
            pltpu.VMEM((128, 32), jnp.float32),
            pltpu.VMEM((128, 32), jnp.float32),
            pltpu.VMEM_SHARED((NP, 32), jnp.float32),
            sem, sem, sem, sem, sem, sem, sem, sem, sem, sem, sem, sem,
        ],
    )(*tables, src_r, dst_r, zeros32)


def _deg_body(dst_r, ones, zeros, out, ib0, ib1, ones_v, acc, s0s, s1s):
    c = lax.axis_index("c")
    t = lax.axis_index("s")
    spt = 20
    pltpu.sync_copy(zeros, acc.at[pl.ds(t * RPT, RPT)])
    pltpu.sync_copy(ones, ones_v)
    plsc.subcore_barrier()
    base = c * 320 + t * spt

    def scat(ib, sem):
        for j in range(10):
            pltpu.async_copy(ones_v, acc.at[ib.at[j]], sem, add=True)

    def swait(ib, sem):
        for j in range(10):
            pltpu.make_async_copy(ones_v, acc.at[ib.at[j]], sem).wait()

    def pair(p, carry):
        @pl.when(p > 0)
        def _():
            swait(ib0, s0s)

        pltpu.sync_copy(dst_r.at[pl.ds((base + 2 * p) * 10, 10)], ib0)
        scat(ib0, s0s)

        @pl.when(p > 0)
        def _():
            swait(ib1, s1s)

        pltpu.sync_copy(dst_r.at[pl.ds((base + 2 * p + 1) * 10, 10)], ib1)
        scat(ib1, s1s)
        return carry

    lax.fori_loop(0, spt // 2, pair, 0)
    swait(ib0, s0s)
    swait(ib1, s1s)
    plsc.subcore_barrier()
    pltpu.sync_copy(acc.at[pl.ds(t * RPT, RPT)], out.at[c].at[pl.ds(t * RPT, RPT)])


def _deg_pass(dst_r, ones8, zeros8):
    return pl.kernel(
        _deg_body,
        out_type=jax.ShapeDtypeStruct((2, NP, 8), jnp.float32),
        mesh=_mesh(),
        compiler_params=_SC_PARAMS,
        scratch_types=[
            pltpu.VMEM((10, 128), jnp.int32),
            pltpu.VMEM((10, 128), jnp.int32),
            pltpu.VMEM((128, 8), jnp.float32),
            pltpu.VMEM_SHARED((NP, 8), jnp.float32),
            pltpu.SemaphoreType.DMA,
            pltpu.SemaphoreType.DMA,
        ],
    )(dst_r, ones8, zeros8)


def _l0_body(x_ref, deg_ref, xsa_ref, xsb_ref, s_ref):
    d = deg_ref[0, :, 0:1] + deg_ref[1, :, 0:1]
    s = lax.rsqrt(1.0 + d)
    xs = x_ref[...] * s
    xsa_ref[0] = xs[:, 0:32]
    xsa_ref[1] = xs[:, 32:64]
    xsb_ref[...] = xs[:, 64:96]
    s_ref[...] = s


def _prep(x_pad, deg):
    return pl.pallas_call(
        _l0_body,
        grid=(16,),
        in_specs=[
            pl.BlockSpec((RPT, 96), lambda i: (i, 0)),
            pl.BlockSpec((2, RPT, 8), lambda i: (0, i, 0)),
        ],
        out_specs=[
            pl.BlockSpec((2, RPT, 32), lambda i: (0, i, 0)),
            pl.BlockSpec((RPT, 32), lambda i: (i, 0)),
            pl.BlockSpec((RPT, 1), lambda i: (i, 0)),
        ],
        out_shape=[
            jax.ShapeDtypeStruct((2, NP, 32), jnp.float32),
            jax.ShapeDtypeStruct((NP, 32), jnp.float32),
            jax.ShapeDtypeStruct((NP, 1), jnp.float32),
        ],
    )(x_pad, deg)


def _u_slices(agg_ref, hsa_ref, hsb_ref, s):
    u0 = (agg_ref[0] + hsa_ref[0]) * s
    u1 = (agg_ref[1] + hsa_ref[1]) * s
    u2 = (agg_ref[2] + agg_ref[3] + hsb_ref[...]) * s
    return (u0, u1, u2)


def _matmul_sliced(us, w_ref, b_ref, width, rows):
    acc = jnp.zeros((rows, width), jnp.float32) + b_ref[...]
    for j, u in enumerate(us):
        acc = acc + jnp.dot(u, w_ref[pl.ds(j * 32, 32)],
                            preferred_element_type=jnp.float32)
    return acc


def _l1_body(agg_ref, hsa_ref, hsb_ref, s_ref, w_ref, b_ref,
             outa_ref, outb_ref):
    i = pl.program_id(0)
    s = s_ref[...]
    us = _u_slices(agg_ref, hsa_ref, hsb_ref, s)
    h = jnp.maximum(_matmul_sliced(us, w_ref, b_ref, 96, RPT), 0.0) * s
    rid = lax.broadcasted_iota(jnp.int32, (RPT, 1), 0) + i * RPT
    h = jnp.where(rid < N, h, 0.0)
    outa_ref[0] = h[:, 0:32]
    outa_ref[1] = h[:, 32:64]
    outb_ref[...] = h[:, 64:96]


def _layer1(agg, hsa, hsb, s, w, b):
    spec4 = pl.BlockSpec((4, RPT, 32), lambda i: (0, i, 0))
    spec2 = pl.BlockSpec((2, RPT, 32), lambda i: (0, i, 0))
    spec1 = pl.BlockSpec((RPT, 32), lambda i: (i, 0))
    return pl.pallas_call(
        _l1_body,
        grid=(16,),
        in_specs=[
            spec4, spec2, spec1,
            pl.BlockSpec((RPT, 1), lambda i: (i, 0)),
            pl.BlockSpec((96, 96), lambda i: (0, 0)),
            pl.BlockSpec((1, 96), lambda i: (0, 0)),
        ],
        out_specs=[spec2, spec1],
        out_shape=[
            jax.ShapeDtypeStruct((2, NP, 32), jnp.float32),
            jax.ShapeDtypeStruct((NP, 32), jnp.float32),
        ],
    )(agg, hsa, hsb, s, w, b)


def _l2_body(agg_ref, hsa_ref, hsb_ref, s_ref, w_ref, b_ref,
             outa1_ref, outa2_ref, outb_ref):
    i = pl.program_id(0)
    s = s_ref[...]
    us = _u_slices(agg_ref, hsa_ref, hsb_ref, s)
    h = jnp.maximum(_matmul_sliced(us, w_ref, b_ref, 160, RPT), 0.0) * s
    rid = lax.broadcasted_iota(jnp.int32, (RPT, 1), 0) + i * RPT
    h = jnp.where(rid < N, h, 0.0)
    outa1_ref[0] = h[:, 0:32]
    outa1_ref[1] = h[:, 32:64]
    outa2_ref[0] = h[:, 64:96]
    outa2_ref[1] = h[:, 96:128]
    outb_ref[...] = h[:, 128:160]


def _layer2(agg, hsa, hsb, s, w, b):
    spec4 = pl.BlockSpec((4, RPT, 32), lambda i: (0, i, 0))
    spec2 = pl.BlockSpec((2, RPT, 32), lambda i: (0, i, 0))
    spec1 = pl.BlockSpec((RPT, 32), lambda i: (i, 0))
    return pl.pallas_call(
        _l2_body,
        grid=(16,),
        in_specs=[
            spec4, spec2, spec1,
            pl.BlockSpec((RPT, 1), lambda i: (i, 0)),
            pl.BlockSpec((96, 160), lambda i: (0, 0)),
            pl.BlockSpec((1, 160), lambda i: (0, 0)),
        ],
        out_specs=[spec2, spec2, spec1],
        out_shape=[
            jax.ShapeDtypeStruct((2, NP, 32), jnp.float32),
            jax.ShapeDtypeStruct((2, NP, 32), jnp.float32),
            jax.ShapeDtypeStruct((NP, 32), jnp.float32),
        ],
    )(agg, hsa, hsb, s, w, b)


PBLK = 400


def _l3_body(batch_sm, agg_ref, h1_ref, h2_ref, hb_ref,
             s_ref, b2d_ref, w_ref, b_ref, out_ref):
    i = pl.program_id(0)

    @pl.when(i == 0)
    def _():
        out_ref[...] = jnp.zeros_like(out_ref)

    s = s_ref[...]
    us = (
        (agg_ref[0] + h1_ref[0]) * s,
        (agg_ref[1] + h1_ref[1]) * s,
        (agg_ref[2] + h2_ref[0]) * s,
        (agg_ref[3] + h2_ref[1]) * s,
        (agg_ref[4] + agg_ref[5] + hb_ref[...]) * s,
    )
    h3 = jnp.maximum(_matmul_sliced(us, w_ref, b_ref, 320, PBLK), 0.0)

    g0 = batch_sm[i * PBLK]
    g1 = batch_sm[i * PBLK + PBLK - 1]
    bids = b2d_ref[...]

    def upd(g, carry):
        gid = g0 + g
        m = (bids == gid)
        v = jnp.max(jnp.where(m, h3, 0.0), axis=0, keepdims=True)
        out_ref[pl.ds(gid, 1), :] = jnp.maximum(out_ref[pl.ds(gid, 1), :], v)
        return carry

    lax.fori_loop(0, g1 - g0 + 1, upd, 0)


def _layer3_pool(batch, agg, h1, h2, hb, s, batch2d, w, b):
    spec6 = pl.BlockSpec((6, PBLK, 32), lambda i, sm: (0, i, 0))
    spec2 = pl.BlockSpec((2, PBLK, 32), lambda i, sm: (0, i, 0))
    spec1 = pl.BlockSpec((PBLK, 32), lambda i, sm: (i, 0))
    grid_spec = pltpu.PrefetchScalarGridSpec(
        num_scalar_prefetch=1,
        grid=(N // PBLK,),
        in_specs=[
            spec6, spec2, spec2, spec1,
            pl.BlockSpec((PBLK, 1), lambda i, sm: (i, 0)),
            pl.BlockSpec((PBLK, 1), lambda i, sm: (i, 0)),
            pl.BlockSpec((160, 320), lambda i, sm: (0, 0)),
            pl.BlockSpec((1, 320), lambda i, sm: (0, 0)),
        ],
        out_specs=pl.BlockSpec((512, 320), lambda i, sm: (0, 0)),
    )
    return pl.pallas_call(
        _l3_body,
        grid_spec=grid_spec,
        out_shape=jax.ShapeDtypeStruct((512, 320), jnp.float32),
    )(batch, agg, h1, h2, hb, s, batch2d, w, b)


def _mlp_body(p_ref, w1_ref, b1_ref, w2_ref, b2_ref, out_ref):
    g = jnp.maximum(jnp.dot(p_ref[...], w1_ref[...],
                            preferred_element_type=jnp.float32) + b1_ref[...], 0.0)
    out_ref[...] = jnp.dot(g, w2_ref[...],
                           preferred_element_type=jnp.float32) + b2_ref[...]


def _mlp(p, w1, b1, w2, b2):
    return pl.pallas_call(
        _mlp_body,
        out_shape=jax.ShapeDtypeStruct((512, 128), jnp.float32),
    )(p, w1, b1, w2, b2)


def kernel(x, edge_index, batch, W1, b1, W2, b2, W3, b3, Wg1, bg1, Wg2, bg2):
    f32 = jnp.float32
    src = edge_index[0].astype(jnp.int32)
    dst = edge_index[1].astype(jnp.int32)
    npad = EPAD - E
    src_r = jnp.concatenate(
        [src, jnp.full((npad,), DUMMY, jnp.int32)]).reshape(EPAD // 128, 128)
    dst_r = jnp.concatenate(
        [dst, jnp.full((npad,), DUMMY, jnp.int32)]).reshape(EPAD // 128, 128)

    zeros32 = jnp.zeros((RPT, 32), f32)
    zeros8 = jnp.zeros((RPT, 8), f32)
    ones8 = jnp.concatenate(
        [jnp.ones((128, 1), f32), jnp.zeros((128, 7), f32)], axis=1)

    x_pad = jnp.zeros((NP, 96), f32).at[:N, :75].set(x)
    W1p = jnp.zeros((96, 96), f32).at[:75, :75].set(W1)
    b1p = jnp.zeros((1, 96), f32).at[0, :75].set(b1)
    W2p = jnp.zeros((96, 160), f32).at[:75, :150].set(W2)
    b2p = jnp.zeros((1, 160), f32).at[0, :150].set(b2)
    W3p = jnp.zeros((160, 320), f32).at[:150, :300].set(W3)
    b3p = jnp.zeros((1, 320), f32).at[0, :300].set(b3)
    Wg1p = jnp.zeros((320, 1024), f32).at[:300].set(Wg1)
    bg1r = bg1.reshape(1, 1024)
    bg2r = bg2.reshape(1, 128)
    batch2d = batch.astype(jnp.int32).reshape(N, 1)

    deg = _deg_pass(dst_r, ones8, zeros8)
    xsa, xsb, s = _prep(x_pad, deg)

    lay12 = [(0, True, 40), (1, False, 20)]
    agg1 = _agg_layer((xsa, xsb), src_r, dst_r, zeros32, phases=lay12)
    h1sa, h1sb = _layer1(agg1, xsa, xsb, s, W1p, b1p)

    agg2 = _agg_layer((h1sa, h1sb), src_r, dst_r, zeros32, phases=lay12)
    h2sa1, h2sa2, h2sb = _layer2(agg2, h1sa, h1sb, s, W2p, b2p)

    agg3 = _agg_layer((h2sa1, h2sa2, h2sb), src_r, dst_r, zeros32,
                      phases=[(0, True, 40), (1, True, 40), (2, False, 20)])

    pooled = _layer3_pool(batch.astype(jnp.int32), agg3,
                          h2sa1, h2sa2, h2sb, s, batch2d, W3p, b3p)
    return _mlp(pooled, Wg1p, bg1r, Wg2, bg2r)
